# baseline (speedup 1.0000x reference)
import functools

import jax
import jax.numpy as jnp
from jax import lax
from jax.experimental import pallas as pl
from jax.experimental.pallas import tpu as pltpu
from jax.experimental.pallas import tpu_sc as plsc

NC = 2
NS = 16
NW = NC * NS
CB = 128
DEGW = 128
GSEG = 64


def _sc_degree(dstflat, zero_deg, acc_n, cpw):
    rows_per_tile = acc_n // NS
    mesh = plsc.VectorSubcoreMesh(core_axis_name="c", subcore_axis_name="s")

    def body(dst_hbm, ones_hbm, zero_hbm, out_hbm,
             dst0, dst1, ones_v, acc_sh, semd0, semd1):
        c = lax.axis_index("c")
        s = lax.axis_index("s")
        w = s * NC + c
        r0 = s * rows_per_tile
        base0 = w * cpw * CB
        pltpu.sync_copy(zero_hbm.at[pl.ds(r0, rows_per_tile)],
                        acc_sh.at[pl.ds(r0, rows_per_tile)])
        pltpu.sync_copy(ones_hbm, ones_v)
        pltpu.async_copy(dst_hbm.at[pl.ds(base0, CB)], dst0, semd0)
        pltpu.async_copy(dst_hbm.at[pl.ds(base0 + CB, CB)], dst1, semd1)
        plsc.subcore_barrier()

        def half(i, dst_a, semd_a):
            pltpu.make_async_copy(dst_hbm.at[pl.ds(0, CB)], dst_a, semd_a).wait()
            pltpu.sync_copy(ones_v, acc_sh.at[dst_a], add=True)
            idd = jnp.minimum(i + 2, cpw - 1)
            pltpu.async_copy(dst_hbm.at[pl.ds(base0 + idd * CB, CB)],
                             dst_a, semd_a)

        def step(t, carry):
            half(2 * t, dst0, semd0)
            half(2 * t + 1, dst1, semd1)
            return carry

        lax.fori_loop(0, cpw // 2, step, 0)
        pltpu.make_async_copy(dst_hbm.at[pl.ds(0, CB)], dst0, semd0).wait()
        pltpu.make_async_copy(dst_hbm.at[pl.ds(0, CB)], dst1, semd1).wait()
        plsc.subcore_barrier()
        pltpu.sync_copy(acc_sh.at[pl.ds(r0, rows_per_tile)],
                        out_hbm.at[c, pl.ds(r0, rows_per_tile)])

    ones = jnp.ones((CB, DEGW), jnp.float32)
    call = pl.kernel(
        body,
        out_type=jax.ShapeDtypeStruct((NC, acc_n, DEGW), jnp.float32),
        mesh=mesh,
        scratch_types=[
            pltpu.VMEM((CB,), jnp.int32),
            pltpu.VMEM((CB,), jnp.int32),
            pltpu.VMEM((CB, DEGW), jnp.float32),
            pltpu.VMEM_SHARED((acc_n, DEGW), jnp.float32),
            pltpu.SemaphoreType.DMA,
            pltpu.SemaphoreType.DMA,
        ],
    )
    return call(dstflat, ones, zero_deg)


def _sc_scatter(ms, src3, dstflat, zero, acc_n, cpw, h):
    rows_per_tile = acc_n // NS
    mesh = plsc.VectorSubcoreMesh(core_axis_name="c", subcore_axis_name="s")

    def body(ms_hbm, src_hbm, dst_hbm, zero_hbm, out_hbm,
             src_v, dst0, dst1, rows0, rows1, acc_sh,
             sem_i, sem0, sem1, semd0, semd1):
        c = lax.axis_index("c")
        s = lax.axis_index("s")
        w = s * NC + c
        r0 = s * rows_per_tile
        base0 = w * cpw * CB
        pltpu.sync_copy(zero_hbm.at[pl.ds(r0, rows_per_tile)],
                        acc_sh.at[pl.ds(r0, rows_per_tile)])
        pltpu.async_copy(src_hbm.at[w], src_v, sem_i).wait()
        pltpu.async_copy(dst_hbm.at[pl.ds(base0, CB)], dst0, semd0)
        pltpu.async_copy(dst_hbm.at[pl.ds(base0 + CB, CB)], dst1, semd1)
        pltpu.async_copy(ms_hbm.at[src_v.at[0]], rows0, sem0)
        plsc.subcore_barrier()

        def half(i, rows_a, sem_a, rows_b, sem_b, dst_a, semd_a):
            pltpu.make_async_copy(ms_hbm.at[pl.ds(0, CB)], rows_a, sem_a).wait()
            inx = jnp.minimum(i + 1, cpw - 1)
            pltpu.async_copy(ms_hbm.at[src_v.at[inx]], rows_b, sem_b)
            pltpu.make_async_copy(dst_hbm.at[pl.ds(0, CB)], dst_a, semd_a).wait()
            pltpu.sync_copy(rows_a, acc_sh.at[dst_a], add=True)
            idd = jnp.minimum(i + 2, cpw - 1)
            pltpu.async_copy(dst_hbm.at[pl.ds(base0 + idd * CB, CB)],
                             dst_a, semd_a)

        def step(t, carry):
            i0 = 2 * t
            half(i0, rows0, sem0, rows1, sem1, dst0, semd0)
            half(i0 + 1, rows1, sem1, rows0, sem0, dst1, semd1)
            return carry

        lax.fori_loop(0, cpw // 2, step, 0)
        pltpu.make_async_copy(ms_hbm.at[pl.ds(0, CB)], rows0, sem0).wait()
        pltpu.make_async_copy(dst_hbm.at[pl.ds(0, CB)], dst0, semd0).wait()
        pltpu.make_async_copy(dst_hbm.at[pl.ds(0, CB)], dst1, semd1).wait()
        plsc.subcore_barrier()
        pltpu.sync_copy(acc_sh.at[pl.ds(r0, rows_per_tile)],
                        out_hbm.at[c, pl.ds(r0, rows_per_tile)])

    call = pl.kernel(
        body,
        out_type=jax.ShapeDtypeStruct((NC, acc_n, h), jnp.float32),
        mesh=mesh,
        scratch_types=[
            pltpu.VMEM((cpw, CB), jnp.int32),
            pltpu.VMEM((CB,), jnp.int32),
            pltpu.VMEM((CB,), jnp.int32),
            pltpu.VMEM((CB, h), jnp.float32),
            pltpu.VMEM((CB, h), jnp.float32),
            pltpu.VMEM_SHARED((acc_n, h), jnp.float32),
            pltpu.SemaphoreType.DMA,
            pltpu.SemaphoreType.DMA,
            pltpu.SemaphoreType.DMA,
            pltpu.SemaphoreType.DMA,
            pltpu.SemaphoreType.DMA,
        ],
    )
    return call(ms, src3, dstflat, zero)


def _tc_init(degp, x, w0, n):
    def body(degp_ref, x_ref, w0_ref, dinv_ref, ms_ref):
        deg = degp_ref[0] + degp_ref[1]
        dinv = lax.rsqrt(deg[:n, 0:1] + 1.0)
        dinv_ref[...] = dinv
        ms_ref[...] = jnp.dot(x_ref[...] * dinv, w0_ref[...],
                              preferred_element_type=jnp.float32)

    h = x.shape[1]
    return pl.pallas_call(
        body,
        out_shape=(jax.ShapeDtypeStruct((n, 1), jnp.float32),
                   jax.ShapeDtypeStruct((n, h), jnp.float32)),
    )(degp, x, w0)


def _layer_post(p_ref, ms_ref, h_ref, dinv_ref, bc_ref, g_ref, b_ref, n):
    ms = ms_ref[...]
    dinv = dinv_ref[...]
    agg = (p_ref[0, :n] + p_ref[1, :n] + ms) * dinv + bc_ref[...]
    a = jnp.maximum(agg, 0.0)
    mu = jnp.mean(a, axis=0, keepdims=True)
    var = jnp.mean((a - mu) ** 2, axis=0, keepdims=True)
    an = (a - mu) * (g_ref[...] * lax.rsqrt(var + 1e-5)) + b_ref[...]
    return an + h_ref[...]


def _tc_layer(p, ms, hprev, dinv, bci, gi, bi, wnext, n):
    def body(p_ref, ms_ref, h_ref, dinv_ref, bc_ref, g_ref, b_ref, wn_ref,
             hout_ref, msout_ref):
        hn = _layer_post(p_ref, ms_ref, h_ref, dinv_ref, bc_ref, g_ref, b_ref, n)
        hout_ref[...] = hn
        msout_ref[...] = jnp.dot(hn * dinv_ref[...], wn_ref[...],
                                 preferred_element_type=jnp.float32)

    h = ms.shape[1]
    return pl.pallas_call(
        body,
        out_shape=(jax.ShapeDtypeStruct((n, h), jnp.float32),
                   jax.ShapeDtypeStruct((n, h), jnp.float32)),
    )(p, ms, hprev, dinv, bci, gi, bi, wnext)


def _tc_final(p, ms, hprev, dinv, bci, gi, bi, batch_row, w1p, b1p, w2p, b2p, n):
    def body(p_ref, ms_ref, h_ref, dinv_ref, bc_ref, g_ref, b_ref,
             batch_ref, w1_ref, b1_ref, w2_ref, b2_ref, out_ref):
        hn = _layer_post(p_ref, ms_ref, h_ref, dinv_ref, bc_ref, g_ref, b_ref, n)
        seg = lax.broadcasted_iota(jnp.int32, (GSEG, n), 0)
        m = (batch_ref[...] == seg).astype(jnp.float32)
        sums = jnp.dot(m, hn, preferred_element_type=jnp.float32)
        cnt = jnp.sum(m, axis=1, keepdims=True)
        pooled = sums / jnp.maximum(cnt, 1.0)
        z = jnp.maximum(
            jnp.dot(pooled, w1_ref[...], preferred_element_type=jnp.float32)
            + b1_ref[...], 0.0)
        out_ref[...] = jnp.dot(z, w2_ref[...],
                               preferred_element_type=jnp.float32) + b2_ref[...]

    return pl.pallas_call(
        body,
        out_shape=jax.ShapeDtypeStruct((GSEG, 128), jnp.float32),
    )(p, ms, hprev, dinv, bci, gi, bi, batch_row, w1p, b1p, w2p, b2p)


def kernel(x, edge_index, batch, Wc, bc, gamma, beta, W1, b1, W2, b2):
    n, h = x.shape
    e = edge_index.shape[1]
    nlayers = Wc.shape[0]
    hh = W1.shape[1]
    nout = W2.shape[1]

    acc_n = -(-(n + 1) // 128) * 128
    cpw = -(-e // (NW * CB))
    cpw += cpw % 2
    ep = NW * CB * cpw
    pad = ep - e
    src3 = jnp.concatenate([edge_index[0], jnp.zeros((pad,), jnp.int32)]
                           ).reshape(NW, cpw, CB)
    dstflat = jnp.concatenate([edge_index[1], jnp.full((pad,), n, jnp.int32)])
    zero_acc = jnp.zeros((acc_n, h), jnp.float32)
    zero_deg = jnp.zeros((acc_n, DEGW), jnp.float32)

    w1p = jnp.zeros((h, 128), jnp.float32).at[:, :hh].set(W1)
    b1p = jnp.zeros((1, 128), jnp.float32).at[0, :hh].set(b1)
    w2p = jnp.zeros((128, 128), jnp.float32).at[:hh, :nout].set(W2)
    b2p = jnp.zeros((1, 128), jnp.float32).at[0, :nout].set(b2)
    batch_row = batch.reshape(1, n)

    degp = _sc_degree(dstflat, zero_deg, acc_n, cpw)
    dinv, ms = _tc_init(degp, x, Wc[0], n)

    hcur = x
    for i in range(nlayers - 1):
        p = _sc_scatter(ms, src3, dstflat, zero_acc, acc_n, cpw, h)
        hcur, ms = _tc_layer(p, ms, hcur, dinv, bc[i].reshape(1, h),
                             gamma[i].reshape(1, h), beta[i].reshape(1, h),
                             Wc[i + 1], n)

    p = _sc_scatter(ms, src3, dstflat, zero_acc, acc_n, cpw, h)
    out_full = _tc_final(p, ms, hcur, dinv, bc[-1].reshape(1, h),
                         gamma[-1].reshape(1, h), beta[-1].reshape(1, h),
                         batch_row, w1p, b1p, w2p, b2p, n)
    return out_full[:, :nout]


# speedup vs baseline: 1.0004x; 1.0004x over previous
import functools

import jax
import jax.numpy as jnp
from jax import lax
from jax.experimental import pallas as pl
from jax.experimental.pallas import tpu as pltpu
from jax.experimental.pallas import tpu_sc as plsc

NC = 2
NS = 16
NW = NC * NS
CB = 128
DEGW = 128
GSEG = 64


def _sc_degree(dstflat, zero_deg, acc_n, cpw):
    rows_per_tile = acc_n // NS
    mesh = plsc.VectorSubcoreMesh(core_axis_name="c", subcore_axis_name="s")

    def body(dst_hbm, ones_hbm, zero_hbm, out_hbm,
             dst0, dst1, ones_v, acc_sh, semd0, semd1):
        c = lax.axis_index("c")
        s = lax.axis_index("s")
        w = s * NC + c
        r0 = s * rows_per_tile
        base0 = w * cpw * CB
        pltpu.sync_copy(zero_hbm.at[pl.ds(r0, rows_per_tile)],
                        acc_sh.at[pl.ds(r0, rows_per_tile)])
        pltpu.sync_copy(ones_hbm, ones_v)
        pltpu.async_copy(dst_hbm.at[pl.ds(base0, CB)], dst0, semd0)
        pltpu.async_copy(dst_hbm.at[pl.ds(base0 + CB, CB)], dst1, semd1)
        plsc.subcore_barrier()

        def half(i, dst_a, semd_a):
            pltpu.make_async_copy(dst_hbm.at[pl.ds(0, CB)], dst_a, semd_a).wait()
            pltpu.sync_copy(ones_v, acc_sh.at[dst_a], add=True)
            idd = jnp.minimum(i + 2, cpw - 1)
            pltpu.async_copy(dst_hbm.at[pl.ds(base0 + idd * CB, CB)],
                             dst_a, semd_a)

        def step(t, carry):
            half(2 * t, dst0, semd0)
            half(2 * t + 1, dst1, semd1)
            return carry

        lax.fori_loop(0, cpw // 2, step, 0)
        pltpu.make_async_copy(dst_hbm.at[pl.ds(0, CB)], dst0, semd0).wait()
        pltpu.make_async_copy(dst_hbm.at[pl.ds(0, CB)], dst1, semd1).wait()
        plsc.subcore_barrier()
        pltpu.sync_copy(acc_sh.at[pl.ds(r0, rows_per_tile)],
                        out_hbm.at[c, pl.ds(r0, rows_per_tile)])

    ones = jnp.ones((CB, DEGW), jnp.float32)
    call = pl.kernel(
        body,
        out_type=jax.ShapeDtypeStruct((NC, acc_n, DEGW), jnp.float32),
        mesh=mesh,
        scratch_types=[
            pltpu.VMEM((CB,), jnp.int32),
            pltpu.VMEM((CB,), jnp.int32),
            pltpu.VMEM((CB, DEGW), jnp.float32),
            pltpu.VMEM_SHARED((acc_n, DEGW), jnp.float32),
            pltpu.SemaphoreType.DMA,
            pltpu.SemaphoreType.DMA,
        ],
    )
    return call(dstflat, ones, zero_deg)


def _sc_scatter(ms, srcflat, dstflat, zero, acc_n, cpw, h):
    rows_per_tile = acc_n // NS
    mesh = plsc.VectorSubcoreMesh(core_axis_name="c", subcore_axis_name="s")

    def body(ms_hbm, src_hbm, dst_hbm, zero_hbm, out_hbm,
             src0, src1, dst0, dst1, rows0, rows1, acc_sh,
             sems0, sems1, semd0, semd1, semr0, semr1):
        c = lax.axis_index("c")
        s = lax.axis_index("s")
        w = s * NC + c
        r0 = s * rows_per_tile
        base0 = w * cpw * CB
        pltpu.sync_copy(zero_hbm.at[pl.ds(r0, rows_per_tile)],
                        acc_sh.at[pl.ds(r0, rows_per_tile)])
        pltpu.async_copy(src_hbm.at[pl.ds(base0, CB)], src0, sems0)
        pltpu.async_copy(src_hbm.at[pl.ds(base0 + CB, CB)], src1, sems1)
        pltpu.async_copy(dst_hbm.at[pl.ds(base0, CB)], dst0, semd0)
        pltpu.async_copy(dst_hbm.at[pl.ds(base0 + CB, CB)], dst1, semd1)
        pltpu.make_async_copy(src_hbm.at[pl.ds(0, CB)], src0, sems0).wait()
        pltpu.async_copy(ms_hbm.at[src0], rows0, semr0)
        plsc.subcore_barrier()

        def half(i, src_a, sems_a, src_b, sems_b, dst_a, semd_a,
                 rows_a, semr_a, rows_b, semr_b):
            pltpu.make_async_copy(ms_hbm.at[pl.ds(0, CB)], rows_a, semr_a).wait()
            inx = jnp.minimum(i + 2, cpw - 1)
            pltpu.async_copy(src_hbm.at[pl.ds(base0 + inx * CB, CB)],
                             src_a, sems_a)
            pltpu.make_async_copy(src_hbm.at[pl.ds(0, CB)], src_b, sems_b).wait()
            pltpu.async_copy(ms_hbm.at[src_b], rows_b, semr_b)
            pltpu.make_async_copy(dst_hbm.at[pl.ds(0, CB)], dst_a, semd_a).wait()
            pltpu.sync_copy(rows_a, acc_sh.at[dst_a], add=True)
            pltpu.async_copy(dst_hbm.at[pl.ds(base0 + inx * CB, CB)],
                             dst_a, semd_a)

        def step(t, carry):
            i0 = 2 * t
            half(i0, src0, sems0, src1, sems1, dst0, semd0,
                 rows0, semr0, rows1, semr1)
            half(i0 + 1, src1, sems1, src0, sems0, dst1, semd1,
                 rows1, semr1, rows0, semr0)
            return carry

        lax.fori_loop(0, cpw // 2, step, 0)
        pltpu.make_async_copy(ms_hbm.at[pl.ds(0, CB)], rows0, semr0).wait()
        pltpu.make_async_copy(src_hbm.at[pl.ds(0, CB)], src1, sems1).wait()
        pltpu.make_async_copy(dst_hbm.at[pl.ds(0, CB)], dst0, semd0).wait()
        pltpu.make_async_copy(dst_hbm.at[pl.ds(0, CB)], dst1, semd1).wait()
        plsc.subcore_barrier()
        pltpu.sync_copy(acc_sh.at[pl.ds(r0, rows_per_tile)],
                        out_hbm.at[c, pl.ds(r0, rows_per_tile)])

    call = pl.kernel(
        body,
        out_type=jax.ShapeDtypeStruct((NC, acc_n, h), jnp.float32),
        mesh=mesh,
        scratch_types=[
            pltpu.VMEM((CB,), jnp.int32),
            pltpu.VMEM((CB,), jnp.int32),
            pltpu.VMEM((CB,), jnp.int32),
            pltpu.VMEM((CB,), jnp.int32),
            pltpu.VMEM((CB, h), jnp.float32),
            pltpu.VMEM((CB, h), jnp.float32),
            pltpu.VMEM_SHARED((acc_n, h), jnp.float32),
            pltpu.SemaphoreType.DMA,
            pltpu.SemaphoreType.DMA,
            pltpu.SemaphoreType.DMA,
            pltpu.SemaphoreType.DMA,
            pltpu.SemaphoreType.DMA,
            pltpu.SemaphoreType.DMA,
        ],
    )
    return call(ms, srcflat, dstflat, zero)


def _tc_init(degp, x, w0, n):
    def body(degp_ref, x_ref, w0_ref, dinv_ref, ms_ref):
        deg = degp_ref[0] + degp_ref[1]
        dinv = lax.rsqrt(deg[:n, 0:1] + 1.0)
        dinv_ref[...] = dinv
        ms_ref[...] = jnp.dot(x_ref[...] * dinv, w0_ref[...],
                              preferred_element_type=jnp.float32)

    h = x.shape[1]
    return pl.pallas_call(
        body,
        out_shape=(jax.ShapeDtypeStruct((n, 1), jnp.float32),
                   jax.ShapeDtypeStruct((n, h), jnp.float32)),
    )(degp, x, w0)


def _layer_post(p_ref, ms_ref, h_ref, dinv_ref, bc_ref, g_ref, b_ref, n):
    ms = ms_ref[...]
    dinv = dinv_ref[...]
    agg = (p_ref[0, :n] + p_ref[1, :n] + ms) * dinv + bc_ref[...]
    a = jnp.maximum(agg, 0.0)
    mu = jnp.mean(a, axis=0, keepdims=True)
    var = jnp.mean((a - mu) ** 2, axis=0, keepdims=True)
    an = (a - mu) * (g_ref[...] * lax.rsqrt(var + 1e-5)) + b_ref[...]
    return an + h_ref[...]


def _tc_layer(p, ms, hprev, dinv, bci, gi, bi, wnext, n):
    def body(p_ref, ms_ref, h_ref, dinv_ref, bc_ref, g_ref, b_ref, wn_ref,
             hout_ref, msout_ref):
        hn = _layer_post(p_ref, ms_ref, h_ref, dinv_ref, bc_ref, g_ref, b_ref, n)
        hout_ref[...] = hn
        msout_ref[...] = jnp.dot(hn * dinv_ref[...], wn_ref[...],
                                 preferred_element_type=jnp.float32)

    h = ms.shape[1]
    return pl.pallas_call(
        body,
        out_shape=(jax.ShapeDtypeStruct((n, h), jnp.float32),
                   jax.ShapeDtypeStruct((n, h), jnp.float32)),
    )(p, ms, hprev, dinv, bci, gi, bi, wnext)


def _tc_final(p, ms, hprev, dinv, bci, gi, bi, batch_row, w1p, b1p, w2p, b2p, n):
    def body(p_ref, ms_ref, h_ref, dinv_ref, bc_ref, g_ref, b_ref,
             batch_ref, w1_ref, b1_ref, w2_ref, b2_ref, out_ref):
        hn = _layer_post(p_ref, ms_ref, h_ref, dinv_ref, bc_ref, g_ref, b_ref, n)
        seg = lax.broadcasted_iota(jnp.int32, (GSEG, n), 0)
        m = (batch_ref[...] == seg).astype(jnp.float32)
        sums = jnp.dot(m, hn, preferred_element_type=jnp.float32)
        cnt = jnp.sum(m, axis=1, keepdims=True)
        pooled = sums / jnp.maximum(cnt, 1.0)
        z = jnp.maximum(
            jnp.dot(pooled, w1_ref[...], preferred_element_type=jnp.float32)
            + b1_ref[...], 0.0)
        out_ref[...] = jnp.dot(z, w2_ref[...],
                               preferred_element_type=jnp.float32) + b2_ref[...]

    return pl.pallas_call(
        body,
        out_shape=jax.ShapeDtypeStruct((GSEG, 128), jnp.float32),
    )(p, ms, hprev, dinv, bci, gi, bi, batch_row, w1p, b1p, w2p, b2p)


def kernel(x, edge_index, batch, Wc, bc, gamma, beta, W1, b1, W2, b2):
    n, h = x.shape
    e = edge_index.shape[1]
    nlayers = Wc.shape[0]
    hh = W1.shape[1]
    nout = W2.shape[1]

    acc_n = -(-(n + 1) // 128) * 128
    cpw = -(-e // (NW * CB))
    cpw += cpw % 2
    ep = NW * CB * cpw
    pad = ep - e
    srcflat = jnp.concatenate([edge_index[0], jnp.zeros((pad,), jnp.int32)])
    dstflat = jnp.concatenate([edge_index[1], jnp.full((pad,), n, jnp.int32)])
    zero_acc = jnp.zeros((acc_n, h), jnp.float32)
    zero_deg = jnp.zeros((acc_n, DEGW), jnp.float32)

    w1p = jnp.zeros((h, 128), jnp.float32).at[:, :hh].set(W1)
    b1p = jnp.zeros((1, 128), jnp.float32).at[0, :hh].set(b1)
    w2p = jnp.zeros((128, 128), jnp.float32).at[:hh, :nout].set(W2)
    b2p = jnp.zeros((1, 128), jnp.float32).at[0, :nout].set(b2)
    batch_row = batch.reshape(1, n)

    degp = _sc_degree(dstflat, zero_deg, acc_n, cpw)
    dinv, ms = _tc_init(degp, x, Wc[0], n)

    hcur = x
    for i in range(nlayers - 1):
        p = _sc_scatter(ms, srcflat, dstflat, zero_acc, acc_n, cpw, h)
        hcur, ms = _tc_layer(p, ms, hcur, dinv, bc[i].reshape(1, h),
                             gamma[i].reshape(1, h), beta[i].reshape(1, h),
                             Wc[i + 1], n)

    p = _sc_scatter(ms, srcflat, dstflat, zero_acc, acc_n, cpw, h)
    out_full = _tc_final(p, ms, hcur, dinv, bc[-1].reshape(1, h),
                         gamma[-1].reshape(1, h), beta[-1].reshape(1, h),
                         batch_row, w1p, b1p, w2p, b2p, n)
    return out_full[:, :nout]


# speedup vs baseline: 1.6327x; 1.6320x over previous
import functools

import jax
import jax.numpy as jnp
from jax import lax
from jax.experimental import pallas as pl
from jax.experimental.pallas import tpu as pltpu
from jax.experimental.pallas import tpu_sc as plsc

NC = 2
NS = 16
NW = NC * NS
CB = 128
CBS = 64
DEGW = 128
GSEG = 64


def _sc_degree(dstflat, zero_deg, acc_n, cpw):
    rows_per_tile = acc_n // NS
    mesh = plsc.VectorSubcoreMesh(core_axis_name="c", subcore_axis_name="s")

    def body(dst_hbm, ones_hbm, zero_hbm, out_hbm,
             dst0, dst1, ones_v, acc_sh, semd0, semd1):
        c = lax.axis_index("c")
        s = lax.axis_index("s")
        w = s * NC + c
        r0 = s * rows_per_tile
        base0 = w * cpw * CB
        pltpu.sync_copy(zero_hbm.at[pl.ds(r0, rows_per_tile)],
                        acc_sh.at[pl.ds(r0, rows_per_tile)])
        pltpu.sync_copy(ones_hbm, ones_v)
        pltpu.async_copy(dst_hbm.at[pl.ds(base0, CB)], dst0, semd0)
        pltpu.async_copy(dst_hbm.at[pl.ds(base0 + CB, CB)], dst1, semd1)
        plsc.subcore_barrier()

        def half(i, dst_a, semd_a):
            pltpu.make_async_copy(dst_hbm.at[pl.ds(0, CB)], dst_a, semd_a).wait()
            pltpu.sync_copy(ones_v, acc_sh.at[dst_a], add=True)
            idd = jnp.minimum(i + 2, cpw - 1)
            pltpu.async_copy(dst_hbm.at[pl.ds(base0 + idd * CB, CB)],
                             dst_a, semd_a)

        def step(t, carry):
            half(2 * t, dst0, semd0)
            half(2 * t + 1, dst1, semd1)
            return carry

        lax.fori_loop(0, cpw // 2, step, 0)
        pltpu.make_async_copy(dst_hbm.at[pl.ds(0, CB)], dst0, semd0).wait()
        pltpu.make_async_copy(dst_hbm.at[pl.ds(0, CB)], dst1, semd1).wait()
        plsc.subcore_barrier()
        pltpu.sync_copy(acc_sh.at[pl.ds(r0, rows_per_tile)],
                        out_hbm.at[c, pl.ds(r0, rows_per_tile)])

    ones = jnp.ones((CB, DEGW), jnp.float32)
    call = pl.kernel(
        body,
        out_type=jax.ShapeDtypeStruct((NC, acc_n, DEGW), jnp.float32),
        mesh=mesh,
        scratch_types=[
            pltpu.VMEM((CB,), jnp.int32),
            pltpu.VMEM((CB,), jnp.int32),
            pltpu.VMEM((CB, DEGW), jnp.float32),
            pltpu.VMEM_SHARED((acc_n, DEGW), jnp.float32),
            pltpu.SemaphoreType.DMA,
            pltpu.SemaphoreType.DMA,
        ],
    )
    return call(dstflat, ones, zero_deg)


def _sc_scatter(ms, src3, dst3, zero, acc_n, cpw, h):
    rows_per_tile = acc_n // NS
    mesh = plsc.VectorSubcoreMesh(core_axis_name="c", subcore_axis_name="s")

    def body(ms_hbm, src_hbm, dst_hbm, zero_hbm, out_hbm,
             src_v, dst0, dst1, dst2, rows0, rows1, rows2,
             acc_sh, sem_i, semd0, semd1, semd2, sem0, sem1, sem2):
        c = lax.axis_index("c")
        s = lax.axis_index("s")
        w = s * NC + c
        r0 = s * rows_per_tile
        base0 = w * cpw * CBS
        pltpu.sync_copy(zero_hbm.at[pl.ds(r0, rows_per_tile)],
                        acc_sh.at[pl.ds(r0, rows_per_tile)])
        pltpu.async_copy(src_hbm.at[w], src_v, sem_i)
        pltpu.async_copy(dst_hbm.at[pl.ds(base0, CBS)], dst0, semd0)
        pltpu.async_copy(dst_hbm.at[pl.ds(base0 + CBS, CBS)], dst1, semd1)
        pltpu.async_copy(dst_hbm.at[pl.ds(base0 + 2 * CBS, CBS)], dst2, semd2)
        pltpu.make_async_copy(src_hbm.at[0], src_v, sem_i).wait()
        pltpu.async_copy(ms_hbm.at[src_v.at[0]], rows0, sem0)
        pltpu.async_copy(ms_hbm.at[src_v.at[1]], rows1, sem1)
        pltpu.async_copy(ms_hbm.at[src_v.at[2]], rows2, sem2)
        plsc.subcore_barrier()

        def third(i, rows_a, sem_a, dst_a, semd_a):
            pltpu.make_async_copy(ms_hbm.at[pl.ds(0, CBS)], rows_a, sem_a).wait()
            pltpu.make_async_copy(dst_hbm.at[pl.ds(0, CBS)], dst_a, semd_a).wait()
            pltpu.sync_copy(rows_a, acc_sh.at[dst_a], add=True)
            inx = jnp.minimum(i + 3, cpw - 1)
            pltpu.async_copy(ms_hbm.at[src_v.at[inx]], rows_a, sem_a)
            pltpu.async_copy(dst_hbm.at[pl.ds(base0 + inx * CBS, CBS)],
                             dst_a, semd_a)

        def step(t, carry):
            i0 = 3 * t
            third(i0, rows0, sem0, dst0, semd0)
            third(i0 + 1, rows1, sem1, dst1, semd1)
            third(i0 + 2, rows2, sem2, dst2, semd2)
            return carry

        lax.fori_loop(0, cpw // 3, step, 0)
        pltpu.make_async_copy(ms_hbm.at[pl.ds(0, CBS)], rows0, sem0).wait()
        pltpu.make_async_copy(ms_hbm.at[pl.ds(0, CBS)], rows1, sem1).wait()
        pltpu.make_async_copy(ms_hbm.at[pl.ds(0, CBS)], rows2, sem2).wait()
        pltpu.make_async_copy(dst_hbm.at[pl.ds(0, CBS)], dst0, semd0).wait()
        pltpu.make_async_copy(dst_hbm.at[pl.ds(0, CBS)], dst1, semd1).wait()
        pltpu.make_async_copy(dst_hbm.at[pl.ds(0, CBS)], dst2, semd2).wait()
        plsc.subcore_barrier()
        pltpu.sync_copy(acc_sh.at[pl.ds(r0, rows_per_tile)],
                        out_hbm.at[c, pl.ds(r0, rows_per_tile)])

    call = pl.kernel(
        body,
        out_type=jax.ShapeDtypeStruct((NC, acc_n, h), jnp.float32),
        mesh=mesh,
        scratch_types=[
            pltpu.VMEM((cpw, CBS), jnp.int32),
            pltpu.VMEM((CBS,), jnp.int32),
            pltpu.VMEM((CBS,), jnp.int32),
            pltpu.VMEM((CBS,), jnp.int32),
            pltpu.VMEM((CBS, h), jnp.float32),
            pltpu.VMEM((CBS, h), jnp.float32),
            pltpu.VMEM((CBS, h), jnp.float32),
            pltpu.VMEM_SHARED((acc_n, h), jnp.float32),
            pltpu.SemaphoreType.DMA,
            pltpu.SemaphoreType.DMA,
            pltpu.SemaphoreType.DMA,
            pltpu.SemaphoreType.DMA,
            pltpu.SemaphoreType.DMA,
            pltpu.SemaphoreType.DMA,
            pltpu.SemaphoreType.DMA,
        ],
    )
    return call(ms, src3, dst3, zero)


def _tc_init(degp, x, w0, n):
    def body(degp_ref, x_ref, w0_ref, dinv_ref, ms_ref):
        deg = degp_ref[0] + degp_ref[1]
        dinv = lax.rsqrt(deg[:n, 0:1] + 1.0)
        dinv_ref[...] = dinv
        ms_ref[...] = jnp.dot(x_ref[...] * dinv, w0_ref[...],
                              preferred_element_type=jnp.float32)

    h = x.shape[1]
    return pl.pallas_call(
        body,
        out_shape=(jax.ShapeDtypeStruct((n, 1), jnp.float32),
                   jax.ShapeDtypeStruct((n, h), jnp.float32)),
    )(degp, x, w0)


def _layer_post(p_ref, ms_ref, h_ref, dinv_ref, bc_ref, g_ref, b_ref, n):
    ms = ms_ref[...]
    dinv = dinv_ref[...]
    agg = (p_ref[0, :n] + p_ref[1, :n] + ms) * dinv + bc_ref[...]
    a = jnp.maximum(agg, 0.0)
    mu = jnp.mean(a, axis=0, keepdims=True)
    var = jnp.mean((a - mu) ** 2, axis=0, keepdims=True)
    an = (a - mu) * (g_ref[...] * lax.rsqrt(var + 1e-5)) + b_ref[...]
    return an + h_ref[...]


def _tc_layer(p, ms, hprev, dinv, bci, gi, bi, wnext, n):
    def body(p_ref, ms_ref, h_ref, dinv_ref, bc_ref, g_ref, b_ref, wn_ref,
             hout_ref, msout_ref):
        hn = _layer_post(p_ref, ms_ref, h_ref, dinv_ref, bc_ref, g_ref, b_ref, n)
        hout_ref[...] = hn
        msout_ref[...] = jnp.dot(hn * dinv_ref[...], wn_ref[...],
                                 preferred_element_type=jnp.float32)

    h = ms.shape[1]
    return pl.pallas_call(
        body,
        out_shape=(jax.ShapeDtypeStruct((n, h), jnp.float32),
                   jax.ShapeDtypeStruct((n, h), jnp.float32)),
    )(p, ms, hprev, dinv, bci, gi, bi, wnext)


def _tc_final(p, ms, hprev, dinv, bci, gi, bi, batch_row, w1p, b1p, w2p, b2p, n):
    def body(p_ref, ms_ref, h_ref, dinv_ref, bc_ref, g_ref, b_ref,
             batch_ref, w1_ref, b1_ref, w2_ref, b2_ref, out_ref):
        hn = _layer_post(p_ref, ms_ref, h_ref, dinv_ref, bc_ref, g_ref, b_ref, n)
        seg = lax.broadcasted_iota(jnp.int32, (GSEG, n), 0)
        m = (batch_ref[...] == seg).astype(jnp.float32)
        sums = jnp.dot(m, hn, preferred_element_type=jnp.float32)
        cnt = jnp.sum(m, axis=1, keepdims=True)
        pooled = sums / jnp.maximum(cnt, 1.0)
        z = jnp.maximum(
            jnp.dot(pooled, w1_ref[...], preferred_element_type=jnp.float32)
            + b1_ref[...], 0.0)
        out_ref[...] = jnp.dot(z, w2_ref[...],
                               preferred_element_type=jnp.float32) + b2_ref[...]

    return pl.pallas_call(
        body,
        out_shape=jax.ShapeDtypeStruct((GSEG, 128), jnp.float32),
    )(p, ms, hprev, dinv, bci, gi, bi, batch_row, w1p, b1p, w2p, b2p)


def kernel(x, edge_index, batch, Wc, bc, gamma, beta, W1, b1, W2, b2):
    n, h = x.shape
    e = edge_index.shape[1]
    nlayers = Wc.shape[0]
    hh = W1.shape[1]
    nout = W2.shape[1]

    acc_n = -(-(n + 1) // 128) * 128
    cpwd = -(-e // (NW * CB))
    cpwd += cpwd % 2
    padd = NW * CB * cpwd - e
    dstflat = jnp.concatenate([edge_index[1], jnp.full((padd,), n, jnp.int32)])
    cpw = -(-e // (NW * CBS))
    cpw += (-cpw) % 3
    pad = NW * CBS * cpw - e
    src3 = jnp.concatenate([edge_index[0], jnp.zeros((pad,), jnp.int32)]
                           ).reshape(NW, cpw, CBS)
    dst3 = jnp.concatenate([edge_index[1], jnp.full((pad,), n, jnp.int32)])
    zero_acc = jnp.zeros((acc_n, h), jnp.float32)
    zero_deg = jnp.zeros((acc_n, DEGW), jnp.float32)

    w1p = jnp.zeros((h, 128), jnp.float32).at[:, :hh].set(W1)
    b1p = jnp.zeros((1, 128), jnp.float32).at[0, :hh].set(b1)
    w2p = jnp.zeros((128, 128), jnp.float32).at[:hh, :nout].set(W2)
    b2p = jnp.zeros((1, 128), jnp.float32).at[0, :nout].set(b2)
    batch_row = batch.reshape(1, n)

    degp = _sc_degree(dstflat, zero_deg, acc_n, cpwd)
    dinv, ms = _tc_init(degp, x, Wc[0], n)

    hcur = x
    for i in range(nlayers - 1):
        p = _sc_scatter(ms, src3, dst3, zero_acc, acc_n, cpw, h)
        hcur, ms = _tc_layer(p, ms, hcur, dinv, bc[i].reshape(1, h),
                             gamma[i].reshape(1, h), beta[i].reshape(1, h),
                             Wc[i + 1], n)

    p = _sc_scatter(ms, src3, dst3, zero_acc, acc_n, cpw, h)
    out_full = _tc_final(p, ms, hcur, dinv, bc[-1].reshape(1, h),
                         gamma[-1].reshape(1, h), beta[-1].reshape(1, h),
                         batch_row, w1p, b1p, w2p, b2p, n)
    return out_full[:, :nout]
